# Initial kernel scaffold; baseline (speedup 1.0000x reference)
import jax
import jax.numpy as jnp
from jax.experimental import pallas as pl


def kernel(atom_types, edges, positions, emb_table, fc1_w, fc1_b, fc2_w, fc2_b, prod_s, prod_v, prod_t):
    raise NotImplementedError("write your pallas kernel here")


# speedup vs baseline: 6.6120x; 6.6120x over previous
import functools

import numpy as np

import jax
import jax.numpy as jnp
from jax import lax
from jax.experimental import pallas as pl
from jax.experimental.pallas import tpu as pltpu
from jax.experimental.pallas import tpu_sc as plsc

N_NODES = 10000
N_EDGES = 160000
EMB = 64
NIRR = 9
ROW = NIRR * EMB
ROWP = 640
PD = 128
NTYPES = 10
R_MAX = 5.0

NC, NS = 2, 16
NW = NC * NS
EB = 128
E_PAD = 163840
NCHK = 250
CHN = 40
OFFP = 272

S3 = 1.7320508
S5 = 2.2360680
S15 = 3.8729833

BE = 512
BN = 1000


def _mesh():
    return plsc.VectorSubcoreMesh(core_axis_name="c", subcore_axis_name="s")


def _lane(vec, i):
    return jnp.sum(jnp.where(lax.iota(jnp.int32, 16) == i, vec, 0))


def _strided_while(start, stop, step, body_fn):
    @pl.loop(start, stop, step=step)
    def _(j):
        body_fn(j)


def _sc_gather(table, idx, d):
    btot = idx.shape[0]
    rows_per_tile = btot // NW
    nb = rows_per_tile // EB

    @functools.partial(
        pl.kernel,
        mesh=_mesh(),
        out_type=jax.ShapeDtypeStruct((btot, d), jnp.float32),
        scratch_types=[
            pltpu.VMEM((EB,), jnp.int32),
            pltpu.VMEM((EB, d), jnp.float32),
            pltpu.SemaphoreType.DMA,
        ],
    )
    def k(table_hbm, idx_hbm, out_hbm, idx_v, rows_v, sem):
        wid = lax.axis_index("s") * NC + lax.axis_index("c")
        base0 = wid * rows_per_tile

        def step(i, carry):
            base = base0 + i * EB
            pltpu.async_copy(idx_hbm.at[pl.ds(base, EB)], idx_v, sem).wait()
            pltpu.async_copy(table_hbm.at[idx_v], rows_v, sem).wait()
            pltpu.async_copy(rows_v, out_hbm.at[pl.ds(base, EB)], sem).wait()
            return carry

        lax.fori_loop(0, nb, step, 0)

    return k(table, idx)


def _sc_scatter(msg, dstp, offs, zrows):
    @functools.partial(
        pl.kernel,
        mesh=_mesh(),
        out_type=jax.ShapeDtypeStruct((N_NODES, ROWP), jnp.float32),
        scratch_types=[
            pltpu.VMEM((EB + 16,), jnp.int32),
            pltpu.VMEM((OFFP,), jnp.int32),
            pltpu.VMEM((EB, ROWP), jnp.float32),
            pltpu.VMEM((CHN, ROWP), jnp.float32),
            pltpu.SemaphoreType.DMA,
        ],
    )
    def k(msg_hbm, dst_hbm, off_hbm, z_hbm, agg_hbm,
          dti_v, off_v, rows_v, acc_v, sem):
        wid = lax.axis_index("s") * NC + lax.axis_index("c")
        pltpu.async_copy(off_hbm, off_v, sem).wait()

        @pl.loop(wid, NCHK, step=NW)
        def _chunk(c):
            base_node = c * CHN
            pltpu.async_copy(z_hbm.at[pl.ds(0, CHN)], acc_v, sem).wait()
            ov = off_v[pl.ds(c, 16)]
            e_lo = ov[0]
            e_hi = ov[1]
            b0 = e_lo // EB
            b1 = (e_hi + (EB - 1)) // EB

            @pl.loop(b0, b1)
            def _batch(b):
                bs = b * EB
                pltpu.async_copy(msg_hbm.at[pl.ds(bs, EB)], rows_v, sem).wait()
                pltpu.async_copy(dst_hbm.at[pl.ds(bs, EB)],
                                 dti_v.at[pl.ds(0, EB)], sem).wait()
                lo_r = jnp.maximum(e_lo - bs, 0)
                hi_r = jnp.minimum(e_hi - bs, EB)

                @pl.loop(lo_r, hi_r)
                def _row(rr):
                    d = dti_v[pl.ds(rr, 16)][0] - base_node
                    for g in range(ROWP // 16):
                        sl = pl.ds(g * 16, 16)
                        acc_v[d, sl] = acc_v[d, sl] + rows_v[rr, sl]

            pltpu.async_copy(acc_v, agg_hbm.at[pl.ds(base_node, CHN)],
                             sem).wait()

    return k(msg, dstp, offs, zrows)


def _embed_body(at_ref, emb_ref, out_ref):
    at = at_ref[...]
    onehot = (at == lax.broadcasted_iota(jnp.int32, (1, NTYPES), 1))
    h0 = jnp.dot(onehot.astype(jnp.float32), emb_ref[...],
                 preferred_element_type=jnp.float32)
    out_ref[...] = jnp.concatenate(
        [h0, jnp.zeros((BN, ROWP - EMB), jnp.float32)], axis=1)


def _embed(atom_types, emb_table):
    return pl.pallas_call(
        _embed_body,
        grid=(N_NODES // BN,),
        in_specs=[pl.BlockSpec((BN, 1), lambda i: (i, 0)),
                  pl.BlockSpec((NTYPES, EMB), lambda i: (0, 0))],
        out_specs=pl.BlockSpec((BN, ROWP), lambda i: (i, 0)),
        out_shape=jax.ShapeDtypeStruct((N_NODES, ROWP), jnp.float32),
    )(atom_types.reshape(N_NODES, 1), emb_table)


def _edge_body(ps_ref, pd_ref, f1_ref, b1_ref, f2_ref, b2_ref,
               sh_ref, w0_ref, w1_ref, w2_ref):
    vec = ps_ref[...] - pd_ref[...]
    r = jnp.sqrt(jnp.sum(vec * vec, axis=1, keepdims=True) + 1e-12)
    u = vec / jnp.maximum(r, 1e-6)
    x, y, z = u[:, 0:1], u[:, 1:2], u[:, 2:3]
    sh_ref[...] = jnp.concatenate(
        [S3 * x, S3 * y, S3 * z,
         S15 * x * y, S15 * y * z, (S5 / 2.0) * (3.0 * z * z - 1.0),
         S15 * x * z, (S15 / 2.0) * (x * x - y * y),
         jnp.zeros((BE, 8), jnp.float32)], axis=1)

    rs = jnp.maximum(r, 1e-6)
    nrow = (lax.broadcasted_iota(jnp.int32, (1, 8), 1) + 1).astype(jnp.float32)
    bes = jnp.sqrt(2.0 / R_MAX) * jnp.sin(nrow * (np.pi / R_MAX) * rs) / rs
    xq = r / R_MAX
    x5 = xq * xq * xq * xq * xq
    env = (1.0 - 21.0 * x5 + 35.0 * x5 * xq - 15.0 * x5 * xq * xq)
    ef = bes * (env * (xq < 1.0))

    f1 = f1_ref[...]
    b1 = b1_ref[...]
    f2 = f2_ref[...]
    b2 = b2_ref[...]
    for i, wref in enumerate((w0_ref, w1_ref, w2_ref)):
        g = jnp.maximum(
            jnp.dot(ef, f1[i], preferred_element_type=jnp.float32)
            + b1[i][None, :], 0.0)
        wref[...] = (jnp.dot(g, f2[i], preferred_element_type=jnp.float32)
                     + b2[i][None, :])


def _edge_prep(pos_src, pos_dst, fc1_w, fc1_b, f2, b2):
    nblk = E_PAD // BE
    full = lambda shp: pl.BlockSpec(shp, lambda i: tuple(0 for _ in shp))
    return pl.pallas_call(
        _edge_body,
        grid=(nblk,),
        in_specs=[pl.BlockSpec((BE, PD), lambda i: (i, 0)),
                  pl.BlockSpec((BE, PD), lambda i: (i, 0)),
                  full((3, 8, EMB)), full((3, EMB)),
                  full((3, EMB, 7 * EMB)), full((3, 7 * EMB))],
        out_specs=[pl.BlockSpec((BE, 16), lambda i: (i, 0)),
                   pl.BlockSpec((BE, 7 * EMB), lambda i: (i, 0)),
                   pl.BlockSpec((BE, 7 * EMB), lambda i: (i, 0)),
                   pl.BlockSpec((BE, 7 * EMB), lambda i: (i, 0))],
        out_shape=[jax.ShapeDtypeStruct((E_PAD, 16), jnp.float32),
                   jax.ShapeDtypeStruct((E_PAD, 7 * EMB), jnp.float32),
                   jax.ShapeDtypeStruct((E_PAD, 7 * EMB), jnp.float32),
                   jax.ShapeDtypeStruct((E_PAD, 7 * EMB), jnp.float32)],
    )(pos_src, pos_dst, fc1_w, fc1_b, f2, b2)


def _msg_body(hs_ref, w_ref, sh_ref, out_ref):
    pid = pl.program_id(0)
    hs = hs_ref[...]
    w = w_ref[...]
    sh = sh_ref[...]
    hj = lambda j: hs[:, j * EMB:(j + 1) * EMB]
    wk = lambda k: w[:, k * EMB:(k + 1) * EMB]
    shc = lambda m: sh[:, m:m + 1]

    h0 = hj(0)
    dot1 = hj(1) * shc(0) + hj(2) * shc(1) + hj(3) * shc(2)
    dot2 = (hj(4) * shc(3) + hj(5) * shc(4) + hj(6) * shc(5)
            + hj(7) * shc(6) + hj(8) * shc(7))
    parts = [wk(0) * h0 + (wk(1) / S3) * dot1 + (wk(2) / S5) * dot2]
    for m in range(3):
        parts.append(wk(3) * h0 * shc(m) + wk(4) * hj(1 + m))
    for m in range(5):
        parts.append(wk(5) * h0 * shc(3 + m) + wk(6) * hj(4 + m))
    parts.append(jnp.zeros((BE, ROWP - ROW), jnp.float32))
    msg = jnp.concatenate(parts, axis=1)
    gid = pid * BE + lax.broadcasted_iota(jnp.int32, (BE, 1), 0)
    out_ref[...] = jnp.where(gid < N_EDGES, msg, 0.0)


def _msg(hs, w, sh16):
    return pl.pallas_call(
        _msg_body,
        grid=(E_PAD // BE,),
        in_specs=[pl.BlockSpec((BE, ROWP), lambda i: (i, 0)),
                  pl.BlockSpec((BE, 7 * EMB), lambda i: (i, 0)),
                  pl.BlockSpec((BE, 16), lambda i: (i, 0))],
        out_specs=pl.BlockSpec((BE, ROWP), lambda i: (i, 0)),
        out_shape=jax.ShapeDtypeStruct((E_PAD, ROWP), jnp.float32),
    )(hs, w, sh16)


def _upd_body(h_ref, agg_ref, s_ref, v_ref, t_ref, out_ref):
    h = h_ref[...]
    agg = agg_ref[...]
    s = s_ref[...]
    v = v_ref[...]
    t = t_ref[...]
    a0 = agg[:, 0:EMB]
    n0 = (h[:, 0:EMB]
          + jnp.dot(a0, s[0], preferred_element_type=jnp.float32)
          + jnp.dot(a0 * a0, s[1], preferred_element_type=jnp.float32)
          + jnp.dot(a0 * a0 * a0, s[2], preferred_element_type=jnp.float32))
    parts = [n0]
    for m in range(1, 4):
        parts.append(h[:, m * EMB:(m + 1) * EMB]
                     + jnp.dot(agg[:, m * EMB:(m + 1) * EMB], v,
                               preferred_element_type=jnp.float32))
    for m in range(4, 9):
        parts.append(h[:, m * EMB:(m + 1) * EMB]
                     + jnp.dot(agg[:, m * EMB:(m + 1) * EMB], t,
                               preferred_element_type=jnp.float32))
    parts.append(jnp.zeros((BN, ROWP - ROW), jnp.float32))
    out_ref[...] = jnp.concatenate(parts, axis=1)


def _update(h, agg, s_i, v_i, t_i):
    full = lambda shp: pl.BlockSpec(shp, lambda i: tuple(0 for _ in shp))
    return pl.pallas_call(
        _upd_body,
        grid=(N_NODES // BN,),
        in_specs=[pl.BlockSpec((BN, ROWP), lambda i: (i, 0)),
                  pl.BlockSpec((BN, ROWP), lambda i: (i, 0)),
                  full((3, EMB, EMB)), full((EMB, EMB)), full((EMB, EMB))],
        out_specs=pl.BlockSpec((BN, ROWP), lambda i: (i, 0)),
        out_shape=jax.ShapeDtypeStruct((N_NODES, ROWP), jnp.float32),
    )(h, agg, s_i, v_i, t_i)


def kernel(atom_types, edges, positions, emb_table, fc1_w, fc1_b,
           fc2_w, fc2_b, prod_s, prod_v, prod_t):
    pos = positions[0]
    src = edges[:, 0].astype(jnp.int32)
    dst = edges[:, 1].astype(jnp.int32)
    order = jnp.argsort(dst)
    src_s = src[order]
    dst_s = dst[order]
    srcp = jnp.zeros((E_PAD,), jnp.int32).at[:N_EDGES].set(src_s)
    dstp = jnp.full((E_PAD,), N_NODES - 1, jnp.int32).at[:N_EDGES].set(dst_s)
    bounds = (jnp.arange(1, NCHK) * CHN).astype(jnp.int32)
    offs_mid = jnp.searchsorted(dst_s, bounds).astype(jnp.int32)
    offs = jnp.concatenate([
        jnp.zeros((1,), jnp.int32), offs_mid,
        jnp.full((OFFP - NCHK,), E_PAD, jnp.int32)])
    pos_pad = jnp.zeros((N_NODES, PD), jnp.float32).at[:, :3].set(pos)
    gidx = jnp.concatenate([srcp, dstp])

    f2 = fc2_w.reshape(3, EMB, EMB, 7).transpose(0, 1, 3, 2).reshape(
        3, EMB, 7 * EMB)
    b2 = fc2_b.reshape(3, EMB, 7).transpose(0, 2, 1).reshape(3, 7 * EMB)

    pos_sd = _sc_gather(pos_pad, gidx, PD)
    sh16, w0, w1, w2 = _edge_prep(pos_sd[:E_PAD], pos_sd[E_PAD:],
                                  fc1_w, fc1_b, f2, b2)
    w_all = (w0, w1, w2)

    zrows = jnp.zeros((EB, ROWP), jnp.float32)
    h = _embed(atom_types.astype(jnp.int32), emb_table)
    for i in range(3):
        hs = _sc_gather(h, srcp, ROWP)
        msg = _msg(hs, w_all[i], sh16)
        agg = _sc_scatter(msg, dstp, offs, zrows)
        h = _update(h, agg, prod_s[i], prod_v[i], prod_t[i])
    return h[:, 0:EMB]


# speedup vs baseline: 6.8796x; 1.0405x over previous
import functools

import numpy as np

import jax
import jax.numpy as jnp
from jax import lax
from jax.experimental import pallas as pl
from jax.experimental.pallas import tpu as pltpu
from jax.experimental.pallas import tpu_sc as plsc

N_NODES = 10000
N_EDGES = 160000
EMB = 64
NIRR = 9
ROW = NIRR * EMB
ROWP = 640
PD = 128
NTYPES = 10
R_MAX = 5.0

NC, NS = 2, 16
NW = NC * NS
EB = 128
E_PAD = 163840
NCHK = 250
CHN = 40
OFFP = 272

S3 = 1.7320508
S5 = 2.2360680
S15 = 3.8729833

BE = 512
BN = 1000


def _mesh():
    return plsc.VectorSubcoreMesh(core_axis_name="c", subcore_axis_name="s")


def _lane(vec, i):
    return jnp.sum(jnp.where(lax.iota(jnp.int32, 16) == i, vec, 0))


def _strided_while(start, stop, step, body_fn):
    @pl.loop(start, stop, step=step)
    def _(j):
        body_fn(j)


def _sc_gather(table, idx, d):
    btot = idx.shape[0]
    rows_per_tile = btot // NW
    gb = 64
    nb = rows_per_tile // gb

    @functools.partial(
        pl.kernel,
        mesh=_mesh(),
        out_type=jax.ShapeDtypeStruct((btot, d), jnp.float32),
        scratch_types=[
            pltpu.VMEM((2, gb), jnp.int32),
            pltpu.VMEM((2, gb, d), jnp.float32),
            pltpu.SemaphoreType.DMA,
            pltpu.SemaphoreType.DMA,
            pltpu.SemaphoreType.DMA,
            pltpu.SemaphoreType.DMA,
            pltpu.SemaphoreType.DMA,
        ],
    )
    def k(table_hbm, idx_hbm, out_hbm, idx_v, rows_v, si, sg0, sg1, so0, so1):
        wid = lax.axis_index("s") * NC + lax.axis_index("c")
        base0 = wid * rows_per_tile
        sg = (sg0, sg1)
        so = (so0, so1)

        def idx_load(i, buf):
            pltpu.async_copy(idx_hbm.at[pl.ds(base0 + i * gb, gb)],
                             idx_v.at[buf], si).wait()

        idx_load(0, 0)
        gd = {0: pltpu.async_copy(table_hbm.at[idx_v.at[0]], rows_v.at[0],
                                  sg[0])}
        od = {}
        for i in range(nb):
            buf = i % 2
            nxt = 1 - buf
            if i + 1 < nb:
                idx_load(i + 1, nxt)
            gd[i].wait()
            if i >= 1:
                od[i - 1].wait()
            od[i] = pltpu.async_copy(rows_v.at[buf],
                                     out_hbm.at[pl.ds(base0 + i * gb, gb)],
                                     so[buf])
            if i + 1 < nb:
                gd[i + 1] = pltpu.async_copy(table_hbm.at[idx_v.at[nxt]],
                                             rows_v.at[nxt], sg[nxt])
        od[nb - 1].wait()

    return k(table, idx)


def _sc_scatter(msg, dstp, offs, zrows):
    @functools.partial(
        pl.kernel,
        mesh=_mesh(),
        out_type=jax.ShapeDtypeStruct((N_NODES, ROWP), jnp.float32),
        scratch_types=[
            pltpu.VMEM((EB + 16,), jnp.int32),
            pltpu.VMEM((OFFP,), jnp.int32),
            pltpu.VMEM((EB, ROWP), jnp.float32),
            pltpu.VMEM((CHN, ROWP), jnp.float32),
            pltpu.SemaphoreType.DMA,
        ],
    )
    def k(msg_hbm, dst_hbm, off_hbm, z_hbm, agg_hbm,
          dti_v, off_v, rows_v, acc_v, sem):
        wid = lax.axis_index("s") * NC + lax.axis_index("c")
        pltpu.async_copy(off_hbm, off_v, sem).wait()

        @pl.loop(wid, NCHK, step=NW)
        def _chunk(c):
            base_node = c * CHN
            pltpu.async_copy(z_hbm.at[pl.ds(0, CHN)], acc_v, sem).wait()
            ov = off_v[pl.ds(c, 16)]
            e_lo = ov[0]
            e_hi = ov[1]
            b0 = e_lo // EB
            b1 = (e_hi + (EB - 1)) // EB

            @pl.loop(b0, b1)
            def _batch(b):
                bs = b * EB
                pltpu.async_copy(msg_hbm.at[pl.ds(bs, EB)], rows_v, sem).wait()
                pltpu.async_copy(dst_hbm.at[pl.ds(bs, EB)],
                                 dti_v.at[pl.ds(0, EB)], sem).wait()
                lo_r = jnp.maximum(e_lo - bs, 0)
                hi_r = jnp.minimum(e_hi - bs, EB)

                @pl.loop(lo_r, hi_r)
                def _row(rr):
                    d = dti_v[pl.ds(rr, 16)][0] - base_node
                    for g in range(ROWP // 16):
                        sl = pl.ds(g * 16, 16)
                        acc_v[d, sl] = acc_v[d, sl] + rows_v[rr, sl]

            pltpu.async_copy(acc_v, agg_hbm.at[pl.ds(base_node, CHN)],
                             sem).wait()

    return k(msg, dstp, offs, zrows)


def _embed_body(at_ref, emb_ref, out_ref):
    at = at_ref[...]
    onehot = (at == lax.broadcasted_iota(jnp.int32, (1, NTYPES), 1))
    h0 = jnp.dot(onehot.astype(jnp.float32), emb_ref[...],
                 preferred_element_type=jnp.float32)
    out_ref[...] = jnp.concatenate(
        [h0, jnp.zeros((BN, ROWP - EMB), jnp.float32)], axis=1)


def _embed0_body(at_ref, emb_ref, out_ref):
    at = at_ref[...]
    onehot = (at == lax.broadcasted_iota(jnp.int32, (1, NTYPES), 1))
    h0 = jnp.dot(onehot.astype(jnp.float32), emb_ref[...],
                 preferred_element_type=jnp.float32)
    out_ref[...] = jnp.concatenate(
        [h0, jnp.zeros((BN, PD - EMB), jnp.float32)], axis=1)


def _embed0(atom_types, emb_table):
    return pl.pallas_call(
        _embed0_body,
        grid=(N_NODES // BN,),
        in_specs=[pl.BlockSpec((BN, 1), lambda i: (i, 0)),
                  pl.BlockSpec((NTYPES, EMB), lambda i: (0, 0))],
        out_specs=pl.BlockSpec((BN, PD), lambda i: (i, 0)),
        out_shape=jax.ShapeDtypeStruct((N_NODES, PD), jnp.float32),
    )(atom_types.reshape(N_NODES, 1), emb_table)


def _embed(atom_types, emb_table):
    return pl.pallas_call(
        _embed_body,
        grid=(N_NODES // BN,),
        in_specs=[pl.BlockSpec((BN, 1), lambda i: (i, 0)),
                  pl.BlockSpec((NTYPES, EMB), lambda i: (0, 0))],
        out_specs=pl.BlockSpec((BN, ROWP), lambda i: (i, 0)),
        out_shape=jax.ShapeDtypeStruct((N_NODES, ROWP), jnp.float32),
    )(atom_types.reshape(N_NODES, 1), emb_table)


def _edge_body(ps_ref, pd_ref, f1_ref, b1_ref, f2_ref, b2_ref,
               sh_ref, w0_ref, w1_ref, w2_ref):
    vec = ps_ref[...] - pd_ref[...]
    r = jnp.sqrt(jnp.sum(vec * vec, axis=1, keepdims=True) + 1e-12)
    u = vec / jnp.maximum(r, 1e-6)
    x, y, z = u[:, 0:1], u[:, 1:2], u[:, 2:3]
    sh_ref[...] = jnp.concatenate(
        [S3 * x, S3 * y, S3 * z,
         S15 * x * y, S15 * y * z, (S5 / 2.0) * (3.0 * z * z - 1.0),
         S15 * x * z, (S15 / 2.0) * (x * x - y * y),
         jnp.zeros((BE, 8), jnp.float32)], axis=1)

    rs = jnp.maximum(r, 1e-6)
    nrow = (lax.broadcasted_iota(jnp.int32, (1, 8), 1) + 1).astype(jnp.float32)
    bes = jnp.sqrt(2.0 / R_MAX) * jnp.sin(nrow * (np.pi / R_MAX) * rs) / rs
    xq = r / R_MAX
    x5 = xq * xq * xq * xq * xq
    env = (1.0 - 21.0 * x5 + 35.0 * x5 * xq - 15.0 * x5 * xq * xq)
    ef = bes * (env * (xq < 1.0))

    f1 = f1_ref[...]
    b1 = b1_ref[...]
    f2 = f2_ref[...]
    b2 = b2_ref[...]
    for i, wref in enumerate((w0_ref, w1_ref, w2_ref)):
        g = jnp.maximum(
            jnp.dot(ef, f1[i], preferred_element_type=jnp.float32)
            + b1[i][None, :], 0.0)
        wref[...] = (jnp.dot(g, f2[i], preferred_element_type=jnp.float32)
                     + b2[i][None, :])


def _edge_prep(pos_src, pos_dst, fc1_w, fc1_b, f2, b2):
    nblk = E_PAD // BE
    full = lambda shp: pl.BlockSpec(shp, lambda i: tuple(0 for _ in shp))
    return pl.pallas_call(
        _edge_body,
        grid=(nblk,),
        in_specs=[pl.BlockSpec((BE, PD), lambda i: (i, 0)),
                  pl.BlockSpec((BE, PD), lambda i: (i, 0)),
                  full((3, 8, EMB)), full((3, EMB)),
                  full((3, EMB, 7 * EMB)), full((3, 7 * EMB))],
        out_specs=[pl.BlockSpec((BE, 16), lambda i: (i, 0)),
                   pl.BlockSpec((BE, 7 * EMB), lambda i: (i, 0)),
                   pl.BlockSpec((BE, 7 * EMB), lambda i: (i, 0)),
                   pl.BlockSpec((BE, 7 * EMB), lambda i: (i, 0))],
        out_shape=[jax.ShapeDtypeStruct((E_PAD, 16), jnp.float32),
                   jax.ShapeDtypeStruct((E_PAD, 7 * EMB), jnp.float32),
                   jax.ShapeDtypeStruct((E_PAD, 7 * EMB), jnp.float32),
                   jax.ShapeDtypeStruct((E_PAD, 7 * EMB), jnp.float32)],
    )(pos_src, pos_dst, fc1_w, fc1_b, f2, b2)


def _msg_body(hs_ref, w_ref, sh_ref, out_ref):
    pid = pl.program_id(0)
    hs = hs_ref[...]
    w = w_ref[...]
    sh = sh_ref[...]
    hj = lambda j: hs[:, j * EMB:(j + 1) * EMB]
    wk = lambda k: w[:, k * EMB:(k + 1) * EMB]
    shc = lambda m: sh[:, m:m + 1]

    h0 = hj(0)
    dot1 = hj(1) * shc(0) + hj(2) * shc(1) + hj(3) * shc(2)
    dot2 = (hj(4) * shc(3) + hj(5) * shc(4) + hj(6) * shc(5)
            + hj(7) * shc(6) + hj(8) * shc(7))
    parts = [wk(0) * h0 + (wk(1) / S3) * dot1 + (wk(2) / S5) * dot2]
    for m in range(3):
        parts.append(wk(3) * h0 * shc(m) + wk(4) * hj(1 + m))
    for m in range(5):
        parts.append(wk(5) * h0 * shc(3 + m) + wk(6) * hj(4 + m))
    parts.append(jnp.zeros((BE, ROWP - ROW), jnp.float32))
    msg = jnp.concatenate(parts, axis=1)
    gid = pid * BE + lax.broadcasted_iota(jnp.int32, (BE, 1), 0)
    out_ref[...] = jnp.where(gid < N_EDGES, msg, 0.0)


def _msg0_body(hs_ref, w_ref, sh_ref, out_ref):
    pid = pl.program_id(0)
    h0 = hs_ref[:, 0:EMB]
    w = w_ref[...]
    sh = sh_ref[...]
    wk = lambda k: w[:, k * EMB:(k + 1) * EMB]
    shc = lambda m: sh[:, m:m + 1]
    parts = [wk(0) * h0]
    for m in range(3):
        parts.append(wk(3) * h0 * shc(m))
    for m in range(5):
        parts.append(wk(5) * h0 * shc(3 + m))
    parts.append(jnp.zeros((BE, ROWP - ROW), jnp.float32))
    msg = jnp.concatenate(parts, axis=1)
    gid = pid * BE + lax.broadcasted_iota(jnp.int32, (BE, 1), 0)
    out_ref[...] = jnp.where(gid < N_EDGES, msg, 0.0)


def _msg0(hs0, w, sh16):
    return pl.pallas_call(
        _msg0_body,
        grid=(E_PAD // BE,),
        in_specs=[pl.BlockSpec((BE, PD), lambda i: (i, 0)),
                  pl.BlockSpec((BE, 7 * EMB), lambda i: (i, 0)),
                  pl.BlockSpec((BE, 16), lambda i: (i, 0))],
        out_specs=pl.BlockSpec((BE, ROWP), lambda i: (i, 0)),
        out_shape=jax.ShapeDtypeStruct((E_PAD, ROWP), jnp.float32),
    )(hs0, w, sh16)


def _msg(hs, w, sh16):
    return pl.pallas_call(
        _msg_body,
        grid=(E_PAD // BE,),
        in_specs=[pl.BlockSpec((BE, ROWP), lambda i: (i, 0)),
                  pl.BlockSpec((BE, 7 * EMB), lambda i: (i, 0)),
                  pl.BlockSpec((BE, 16), lambda i: (i, 0))],
        out_specs=pl.BlockSpec((BE, ROWP), lambda i: (i, 0)),
        out_shape=jax.ShapeDtypeStruct((E_PAD, ROWP), jnp.float32),
    )(hs, w, sh16)


def _upd_body(h_ref, agg_ref, s_ref, v_ref, t_ref, out_ref):
    h = h_ref[...]
    agg = agg_ref[...]
    s = s_ref[...]
    v = v_ref[...]
    t = t_ref[...]
    a0 = agg[:, 0:EMB]
    n0 = (h[:, 0:EMB]
          + jnp.dot(a0, s[0], preferred_element_type=jnp.float32)
          + jnp.dot(a0 * a0, s[1], preferred_element_type=jnp.float32)
          + jnp.dot(a0 * a0 * a0, s[2], preferred_element_type=jnp.float32))
    parts = [n0]
    for m in range(1, 4):
        parts.append(h[:, m * EMB:(m + 1) * EMB]
                     + jnp.dot(agg[:, m * EMB:(m + 1) * EMB], v,
                               preferred_element_type=jnp.float32))
    for m in range(4, 9):
        parts.append(h[:, m * EMB:(m + 1) * EMB]
                     + jnp.dot(agg[:, m * EMB:(m + 1) * EMB], t,
                               preferred_element_type=jnp.float32))
    parts.append(jnp.zeros((BN, ROWP - ROW), jnp.float32))
    out_ref[...] = jnp.concatenate(parts, axis=1)


def _update(h, agg, s_i, v_i, t_i):
    full = lambda shp: pl.BlockSpec(shp, lambda i: tuple(0 for _ in shp))
    return pl.pallas_call(
        _upd_body,
        grid=(N_NODES // BN,),
        in_specs=[pl.BlockSpec((BN, ROWP), lambda i: (i, 0)),
                  pl.BlockSpec((BN, ROWP), lambda i: (i, 0)),
                  full((3, EMB, EMB)), full((EMB, EMB)), full((EMB, EMB))],
        out_specs=pl.BlockSpec((BN, ROWP), lambda i: (i, 0)),
        out_shape=jax.ShapeDtypeStruct((N_NODES, ROWP), jnp.float32),
    )(h, agg, s_i, v_i, t_i)


def kernel(atom_types, edges, positions, emb_table, fc1_w, fc1_b,
           fc2_w, fc2_b, prod_s, prod_v, prod_t):
    pos = positions[0]
    src = edges[:, 0].astype(jnp.int32)
    dst = edges[:, 1].astype(jnp.int32)
    order = jnp.argsort(dst)
    src_s = src[order]
    dst_s = dst[order]
    srcp = jnp.zeros((E_PAD,), jnp.int32).at[:N_EDGES].set(src_s)
    dstp = jnp.full((E_PAD,), N_NODES - 1, jnp.int32).at[:N_EDGES].set(dst_s)
    bounds = (jnp.arange(1, NCHK) * CHN).astype(jnp.int32)
    offs_mid = jnp.searchsorted(dst_s, bounds).astype(jnp.int32)
    offs = jnp.concatenate([
        jnp.zeros((1,), jnp.int32), offs_mid,
        jnp.full((OFFP - NCHK,), E_PAD, jnp.int32)])
    pos_pad = jnp.zeros((N_NODES, PD), jnp.float32).at[:, :3].set(pos)
    gidx = jnp.concatenate([srcp, dstp])

    f2 = fc2_w.reshape(3, EMB, EMB, 7).transpose(0, 1, 3, 2).reshape(
        3, EMB, 7 * EMB)
    b2 = fc2_b.reshape(3, EMB, 7).transpose(0, 2, 1).reshape(3, 7 * EMB)

    pos_sd = _sc_gather(pos_pad, gidx, PD)
    sh16, w0, w1, w2 = _edge_prep(pos_sd[:E_PAD], pos_sd[E_PAD:],
                                  fc1_w, fc1_b, f2, b2)
    w_all = (w0, w1, w2)

    zrows = jnp.zeros((EB, ROWP), jnp.float32)
    at32 = atom_types.astype(jnp.int32)
    h = _embed(at32, emb_table)
    h0_tab = _embed0(at32, emb_table)
    for i in range(3):
        if i == 0:
            hs0 = _sc_gather(h0_tab, srcp, PD)
            msg = _msg0(hs0, w_all[i], sh16)
        else:
            hs = _sc_gather(h, srcp, ROWP)
            msg = _msg(hs, w_all[i], sh16)
        agg = _sc_scatter(msg, dstp, offs, zrows)
        h = _update(h, agg, prod_s[i], prod_v[i], prod_t[i])
    return h[:, 0:EMB]


# speedup vs baseline: 7.0968x; 1.0316x over previous
import functools

import numpy as np

import jax
import jax.numpy as jnp
from jax import lax
from jax.experimental import pallas as pl
from jax.experimental.pallas import tpu as pltpu
from jax.experimental.pallas import tpu_sc as plsc

N_NODES = 10000
N_EDGES = 160000
EMB = 64
NIRR = 9
ROW = NIRR * EMB
ROWP = 640
PD = 128
NTYPES = 10
R_MAX = 5.0

NC, NS = 2, 16
NW = NC * NS
EB = 128
E_PAD = 163840
NCHK = 250
CHN = 40
OFFP = 272

S3 = 1.7320508
S5 = 2.2360680
S15 = 3.8729833

BE = 512
BN = 1000


def _mesh():
    return plsc.VectorSubcoreMesh(core_axis_name="c", subcore_axis_name="s")


def _lane(vec, i):
    return jnp.sum(jnp.where(lax.iota(jnp.int32, 16) == i, vec, 0))


def _strided_while(start, stop, step, body_fn):
    @pl.loop(start, stop, step=step)
    def _(j):
        body_fn(j)


def _sc_gather(table, idx, d):
    btot = idx.shape[0]
    rows_per_tile = btot // NW
    gb = 64
    nb = rows_per_tile // gb

    @functools.partial(
        pl.kernel,
        mesh=_mesh(),
        out_type=jax.ShapeDtypeStruct((btot, d), jnp.float32),
        scratch_types=[
            pltpu.VMEM((2, gb), jnp.int32),
            pltpu.VMEM((2, gb, d), jnp.float32),
            pltpu.SemaphoreType.DMA,
            pltpu.SemaphoreType.DMA,
            pltpu.SemaphoreType.DMA,
            pltpu.SemaphoreType.DMA,
            pltpu.SemaphoreType.DMA,
        ],
    )
    def k(table_hbm, idx_hbm, out_hbm, idx_v, rows_v, si, sg0, sg1, so0, so1):
        wid = lax.axis_index("s") * NC + lax.axis_index("c")
        base0 = wid * rows_per_tile
        sg = (sg0, sg1)
        so = (so0, so1)

        def idx_load(i, buf):
            pltpu.async_copy(idx_hbm.at[pl.ds(base0 + i * gb, gb)],
                             idx_v.at[buf], si).wait()

        idx_load(0, 0)
        gd = {0: pltpu.async_copy(table_hbm.at[idx_v.at[0]], rows_v.at[0],
                                  sg[0])}
        od = {}
        for i in range(nb):
            buf = i % 2
            nxt = 1 - buf
            if i + 1 < nb:
                idx_load(i + 1, nxt)
            gd[i].wait()
            if i >= 1:
                od[i - 1].wait()
            od[i] = pltpu.async_copy(rows_v.at[buf],
                                     out_hbm.at[pl.ds(base0 + i * gb, gb)],
                                     so[buf])
            if i + 1 < nb:
                gd[i + 1] = pltpu.async_copy(table_hbm.at[idx_v.at[nxt]],
                                             rows_v.at[nxt], sg[nxt])
        od[nb - 1].wait()

    return k(table, idx)


def _sc_scatter(msg, dstp, offs, zrows):
    EBS = 64

    @functools.partial(
        pl.kernel,
        mesh=_mesh(),
        out_type=jax.ShapeDtypeStruct((N_NODES, ROWP), jnp.float32),
        scratch_types=[
            pltpu.VMEM((EBS + 16,), jnp.int32),
            pltpu.VMEM((EBS + 16,), jnp.int32),
            pltpu.VMEM((OFFP,), jnp.int32),
            pltpu.VMEM((EBS, ROWP), jnp.float32),
            pltpu.VMEM((EBS, ROWP), jnp.float32),
            pltpu.VMEM((CHN + 1, ROWP), jnp.float32),
            pltpu.SemaphoreType.DMA,
            pltpu.SemaphoreType.DMA,
            pltpu.SemaphoreType.DMA,
            pltpu.SemaphoreType.DMA,
            pltpu.SemaphoreType.DMA,
        ],
    )
    def k(msg_hbm, dst_hbm, off_hbm, z_hbm, agg_hbm,
          dti_a, dti_b, off_v, rows_a, rows_b, acc_v, s0, s1, sd0, sd1, sz):
        wid = lax.axis_index("s") * NC + lax.axis_index("c")
        pltpu.async_copy(off_hbm, off_v, sz).wait()
        zero16 = jnp.zeros((16,), jnp.float32)
        ngrp = ROWP // 16

        @pl.loop(wid, NCHK, step=NW)
        def _chunk(c):
            base_node = c * CHN
            pltpu.async_copy(z_hbm.at[pl.ds(0, CHN)],
                             acc_v.at[pl.ds(0, CHN)], sz).wait()
            ov = off_v[pl.ds(c, 16)]
            e_lo = ov[0]
            e_hi = ov[1]
            b0 = e_lo // EBS
            b1 = (e_hi + (EBS - 1)) // EBS

            def process(bs, dti_v, rows_v, carry):
                lo_r = jnp.maximum(e_lo - bs, 0)
                hi_r = jnp.minimum(e_hi - bs, EBS)

                @pl.loop(lo_r, hi_r, init_carry=carry)
                def _row(rr, rcarry):
                    rregs, rcur = rcarry
                    d = dti_v[pl.ds(rr, 16)][0] - base_node
                    keep = 1.0 - (d != rcur).astype(jnp.float32)
                    new = []
                    for g in range(ngrp):
                        row = rows_v[rr, pl.ds(g * 16, 16)]
                        acc = rregs[g] * keep + row
                        acc_v[d, pl.ds(g * 16, 16)] = acc
                        new.append(acc)
                    return (tuple(new), d)

                return _row if _row is not None else carry

            init = (tuple(zero16 for _ in range(ngrp)), jnp.int32(CHN))

            @pl.loop(b0, b1, step=2, init_carry=init)
            def _batch(b, carry):
                bs_a = b * EBS
                bs_br = bs_a + EBS
                bs_b = jnp.minimum(bs_br, E_PAD - EBS)
                da = pltpu.async_copy(msg_hbm.at[pl.ds(bs_a, EBS)],
                                      rows_a, s0)
                dda = pltpu.async_copy(dst_hbm.at[pl.ds(bs_a, EBS)],
                                       dti_a.at[pl.ds(0, EBS)], sd0)
                db = pltpu.async_copy(msg_hbm.at[pl.ds(bs_b, EBS)],
                                      rows_b, s1)
                ddb = pltpu.async_copy(dst_hbm.at[pl.ds(bs_b, EBS)],
                                       dti_b.at[pl.ds(0, EBS)], sd1)
                da.wait()
                dda.wait()
                carry = process(bs_a, dti_a, rows_a, carry)
                db.wait()
                ddb.wait()
                carry = process(bs_br, dti_b, rows_b, carry)
                return carry

            pltpu.async_copy(acc_v.at[pl.ds(0, CHN)],
                             agg_hbm.at[pl.ds(base_node, CHN)], sz).wait()

    return k(msg, dstp, offs, zrows)


def _embed_body(at_ref, emb_ref, out_ref):
    at = at_ref[...]
    onehot = (at == lax.broadcasted_iota(jnp.int32, (1, NTYPES), 1))
    h0 = jnp.dot(onehot.astype(jnp.float32), emb_ref[...],
                 preferred_element_type=jnp.float32)
    out_ref[...] = jnp.concatenate(
        [h0, jnp.zeros((BN, ROWP - EMB), jnp.float32)], axis=1)


def _embed0_body(at_ref, emb_ref, out_ref):
    at = at_ref[...]
    onehot = (at == lax.broadcasted_iota(jnp.int32, (1, NTYPES), 1))
    h0 = jnp.dot(onehot.astype(jnp.float32), emb_ref[...],
                 preferred_element_type=jnp.float32)
    out_ref[...] = jnp.concatenate(
        [h0, jnp.zeros((BN, PD - EMB), jnp.float32)], axis=1)


def _embed0(atom_types, emb_table):
    return pl.pallas_call(
        _embed0_body,
        grid=(N_NODES // BN,),
        in_specs=[pl.BlockSpec((BN, 1), lambda i: (i, 0)),
                  pl.BlockSpec((NTYPES, EMB), lambda i: (0, 0))],
        out_specs=pl.BlockSpec((BN, PD), lambda i: (i, 0)),
        out_shape=jax.ShapeDtypeStruct((N_NODES, PD), jnp.float32),
    )(atom_types.reshape(N_NODES, 1), emb_table)


def _embed(atom_types, emb_table):
    return pl.pallas_call(
        _embed_body,
        grid=(N_NODES // BN,),
        in_specs=[pl.BlockSpec((BN, 1), lambda i: (i, 0)),
                  pl.BlockSpec((NTYPES, EMB), lambda i: (0, 0))],
        out_specs=pl.BlockSpec((BN, ROWP), lambda i: (i, 0)),
        out_shape=jax.ShapeDtypeStruct((N_NODES, ROWP), jnp.float32),
    )(atom_types.reshape(N_NODES, 1), emb_table)


def _edge_body(ps_ref, pd_ref, f1_ref, b1_ref, f2_ref, b2_ref,
               sh_ref, w0_ref, w1_ref, w2_ref):
    vec = ps_ref[...] - pd_ref[...]
    r = jnp.sqrt(jnp.sum(vec * vec, axis=1, keepdims=True) + 1e-12)
    u = vec / jnp.maximum(r, 1e-6)
    x, y, z = u[:, 0:1], u[:, 1:2], u[:, 2:3]
    sh_ref[...] = jnp.concatenate(
        [S3 * x, S3 * y, S3 * z,
         S15 * x * y, S15 * y * z, (S5 / 2.0) * (3.0 * z * z - 1.0),
         S15 * x * z, (S15 / 2.0) * (x * x - y * y),
         jnp.zeros((BE, 8), jnp.float32)], axis=1)

    rs = jnp.maximum(r, 1e-6)
    nrow = (lax.broadcasted_iota(jnp.int32, (1, 8), 1) + 1).astype(jnp.float32)
    bes = jnp.sqrt(2.0 / R_MAX) * jnp.sin(nrow * (np.pi / R_MAX) * rs) / rs
    xq = r / R_MAX
    x5 = xq * xq * xq * xq * xq
    env = (1.0 - 21.0 * x5 + 35.0 * x5 * xq - 15.0 * x5 * xq * xq)
    ef = bes * (env * (xq < 1.0))

    f1 = f1_ref[...]
    b1 = b1_ref[...]
    f2 = f2_ref[...]
    b2 = b2_ref[...]
    for i, wref in enumerate((w0_ref, w1_ref, w2_ref)):
        g = jnp.maximum(
            jnp.dot(ef, f1[i], preferred_element_type=jnp.float32)
            + b1[i][None, :], 0.0)
        wref[...] = (jnp.dot(g, f2[i], preferred_element_type=jnp.float32)
                     + b2[i][None, :])


def _edge_prep(pos_src, pos_dst, fc1_w, fc1_b, f2, b2):
    nblk = E_PAD // BE
    full = lambda shp: pl.BlockSpec(shp, lambda i: tuple(0 for _ in shp))
    return pl.pallas_call(
        _edge_body,
        grid=(nblk,),
        in_specs=[pl.BlockSpec((BE, PD), lambda i: (i, 0)),
                  pl.BlockSpec((BE, PD), lambda i: (i, 0)),
                  full((3, 8, EMB)), full((3, EMB)),
                  full((3, EMB, 7 * EMB)), full((3, 7 * EMB))],
        out_specs=[pl.BlockSpec((BE, 16), lambda i: (i, 0)),
                   pl.BlockSpec((BE, 7 * EMB), lambda i: (i, 0)),
                   pl.BlockSpec((BE, 7 * EMB), lambda i: (i, 0)),
                   pl.BlockSpec((BE, 7 * EMB), lambda i: (i, 0))],
        out_shape=[jax.ShapeDtypeStruct((E_PAD, 16), jnp.float32),
                   jax.ShapeDtypeStruct((E_PAD, 7 * EMB), jnp.float32),
                   jax.ShapeDtypeStruct((E_PAD, 7 * EMB), jnp.float32),
                   jax.ShapeDtypeStruct((E_PAD, 7 * EMB), jnp.float32)],
    )(pos_src, pos_dst, fc1_w, fc1_b, f2, b2)


def _msg_body(hs_ref, w_ref, sh_ref, out_ref):
    pid = pl.program_id(0)
    hs = hs_ref[...]
    w = w_ref[...]
    sh = sh_ref[...]
    hj = lambda j: hs[:, j * EMB:(j + 1) * EMB]
    wk = lambda k: w[:, k * EMB:(k + 1) * EMB]
    shc = lambda m: sh[:, m:m + 1]

    h0 = hj(0)
    dot1 = hj(1) * shc(0) + hj(2) * shc(1) + hj(3) * shc(2)
    dot2 = (hj(4) * shc(3) + hj(5) * shc(4) + hj(6) * shc(5)
            + hj(7) * shc(6) + hj(8) * shc(7))
    parts = [wk(0) * h0 + (wk(1) / S3) * dot1 + (wk(2) / S5) * dot2]
    for m in range(3):
        parts.append(wk(3) * h0 * shc(m) + wk(4) * hj(1 + m))
    for m in range(5):
        parts.append(wk(5) * h0 * shc(3 + m) + wk(6) * hj(4 + m))
    parts.append(jnp.zeros((BE, ROWP - ROW), jnp.float32))
    msg = jnp.concatenate(parts, axis=1)
    gid = pid * BE + lax.broadcasted_iota(jnp.int32, (BE, 1), 0)
    out_ref[...] = jnp.where(gid < N_EDGES, msg, 0.0)


def _msg0_body(hs_ref, w_ref, sh_ref, out_ref):
    pid = pl.program_id(0)
    h0 = hs_ref[:, 0:EMB]
    w = w_ref[...]
    sh = sh_ref[...]
    wk = lambda k: w[:, k * EMB:(k + 1) * EMB]
    shc = lambda m: sh[:, m:m + 1]
    parts = [wk(0) * h0]
    for m in range(3):
        parts.append(wk(3) * h0 * shc(m))
    for m in range(5):
        parts.append(wk(5) * h0 * shc(3 + m))
    parts.append(jnp.zeros((BE, ROWP - ROW), jnp.float32))
    msg = jnp.concatenate(parts, axis=1)
    gid = pid * BE + lax.broadcasted_iota(jnp.int32, (BE, 1), 0)
    out_ref[...] = jnp.where(gid < N_EDGES, msg, 0.0)


def _msg0(hs0, w, sh16):
    return pl.pallas_call(
        _msg0_body,
        grid=(E_PAD // BE,),
        in_specs=[pl.BlockSpec((BE, PD), lambda i: (i, 0)),
                  pl.BlockSpec((BE, 7 * EMB), lambda i: (i, 0)),
                  pl.BlockSpec((BE, 16), lambda i: (i, 0))],
        out_specs=pl.BlockSpec((BE, ROWP), lambda i: (i, 0)),
        out_shape=jax.ShapeDtypeStruct((E_PAD, ROWP), jnp.float32),
    )(hs0, w, sh16)


def _msg(hs, w, sh16):
    return pl.pallas_call(
        _msg_body,
        grid=(E_PAD // BE,),
        in_specs=[pl.BlockSpec((BE, ROWP), lambda i: (i, 0)),
                  pl.BlockSpec((BE, 7 * EMB), lambda i: (i, 0)),
                  pl.BlockSpec((BE, 16), lambda i: (i, 0))],
        out_specs=pl.BlockSpec((BE, ROWP), lambda i: (i, 0)),
        out_shape=jax.ShapeDtypeStruct((E_PAD, ROWP), jnp.float32),
    )(hs, w, sh16)


def _upd_body(h_ref, agg_ref, s_ref, v_ref, t_ref, out_ref):
    h = h_ref[...]
    agg = agg_ref[...]
    s = s_ref[...]
    v = v_ref[...]
    t = t_ref[...]
    a0 = agg[:, 0:EMB]
    n0 = (h[:, 0:EMB]
          + jnp.dot(a0, s[0], preferred_element_type=jnp.float32)
          + jnp.dot(a0 * a0, s[1], preferred_element_type=jnp.float32)
          + jnp.dot(a0 * a0 * a0, s[2], preferred_element_type=jnp.float32))
    parts = [n0]
    for m in range(1, 4):
        parts.append(h[:, m * EMB:(m + 1) * EMB]
                     + jnp.dot(agg[:, m * EMB:(m + 1) * EMB], v,
                               preferred_element_type=jnp.float32))
    for m in range(4, 9):
        parts.append(h[:, m * EMB:(m + 1) * EMB]
                     + jnp.dot(agg[:, m * EMB:(m + 1) * EMB], t,
                               preferred_element_type=jnp.float32))
    parts.append(jnp.zeros((BN, ROWP - ROW), jnp.float32))
    out_ref[...] = jnp.concatenate(parts, axis=1)


def _update(h, agg, s_i, v_i, t_i):
    full = lambda shp: pl.BlockSpec(shp, lambda i: tuple(0 for _ in shp))
    return pl.pallas_call(
        _upd_body,
        grid=(N_NODES // BN,),
        in_specs=[pl.BlockSpec((BN, ROWP), lambda i: (i, 0)),
                  pl.BlockSpec((BN, ROWP), lambda i: (i, 0)),
                  full((3, EMB, EMB)), full((EMB, EMB)), full((EMB, EMB))],
        out_specs=pl.BlockSpec((BN, ROWP), lambda i: (i, 0)),
        out_shape=jax.ShapeDtypeStruct((N_NODES, ROWP), jnp.float32),
    )(h, agg, s_i, v_i, t_i)


def kernel(atom_types, edges, positions, emb_table, fc1_w, fc1_b,
           fc2_w, fc2_b, prod_s, prod_v, prod_t):
    pos = positions[0]
    src = edges[:, 0].astype(jnp.int32)
    dst = edges[:, 1].astype(jnp.int32)
    order = jnp.argsort(dst)
    src_s = src[order]
    dst_s = dst[order]
    srcp = jnp.zeros((E_PAD,), jnp.int32).at[:N_EDGES].set(src_s)
    dstp = jnp.full((E_PAD,), N_NODES - 1, jnp.int32).at[:N_EDGES].set(dst_s)
    bounds = (jnp.arange(1, NCHK) * CHN).astype(jnp.int32)
    offs_mid = jnp.searchsorted(dst_s, bounds).astype(jnp.int32)
    offs = jnp.concatenate([
        jnp.zeros((1,), jnp.int32), offs_mid,
        jnp.full((OFFP - NCHK,), E_PAD, jnp.int32)])
    pos_pad = jnp.zeros((N_NODES, PD), jnp.float32).at[:, :3].set(pos)
    gidx = jnp.concatenate([srcp, dstp])

    f2 = fc2_w.reshape(3, EMB, EMB, 7).transpose(0, 1, 3, 2).reshape(
        3, EMB, 7 * EMB)
    b2 = fc2_b.reshape(3, EMB, 7).transpose(0, 2, 1).reshape(3, 7 * EMB)

    pos_sd = _sc_gather(pos_pad, gidx, PD)
    sh16, w0, w1, w2 = _edge_prep(pos_sd[:E_PAD], pos_sd[E_PAD:],
                                  fc1_w, fc1_b, f2, b2)
    w_all = (w0, w1, w2)

    zrows = jnp.zeros((EB, ROWP), jnp.float32)
    at32 = atom_types.astype(jnp.int32)
    h = _embed(at32, emb_table)
    h0_tab = _embed0(at32, emb_table)
    for i in range(3):
        if i == 0:
            hs0 = _sc_gather(h0_tab, srcp, PD)
            msg = _msg0(hs0, w_all[i], sh16)
        else:
            hs = _sc_gather(h, srcp, ROWP)
            msg = _msg(hs, w_all[i], sh16)
        agg = _sc_scatter(msg, dstp, offs, zrows)
        h = _update(h, agg, prod_s[i], prod_v[i], prod_t[i])
    return h[:, 0:EMB]
